# Initial kernel scaffold; baseline (speedup 1.0000x reference)
import jax
import jax.numpy as jnp
from jax.experimental import pallas as pl


def kernel(h_e, x_orig, Wq, bq, Wk, bk, Wv, bv, W1, b1, W2, b2):
    raise NotImplementedError("write your pallas kernel here")


# speedup vs baseline: 12.6882x; 12.6882x over previous
import functools
import math

import jax
import jax.numpy as jnp
from jax.experimental import pallas as pl

_BATCH = 16
_N = 512
_D = 128
_TAU = 3
_K = 8
_BIG = 3.0e38


def _dotT(a, b):
    return jax.lax.dot_general(a, b, (((1,), (1,)), ((), ())),
                               preferred_element_type=jnp.float32)


def _dot(a, b):
    return jax.lax.dot_general(a, b, (((1,), (0,)), ((), ())),
                               preferred_element_type=jnp.float32)


def _erf(x):
    a1, a2, a3 = 0.254829592, -0.284496736, 1.421413741
    a4, a5, p = -1.453152027, 1.061405429, 0.3275911
    s = jnp.sign(x)
    ax = jnp.abs(x)
    t = 1.0 / (1.0 + p * ax)
    y = 1.0 - (((((a5 * t + a4) * t) + a3) * t + a2) * t + a1) * t * jnp.exp(-ax * ax)
    return s * y


def _body(hist_ref, wcol_ref, wrow_ref,
          wq_ref, wk_ref, wv_ref, w1_ref, w2_ref,
          bq_ref, bk_ref, bv_ref, b1_ref, b2_ref,
          out_ref):
    hist = hist_ref[0]
    he_last = hist[(_TAU - 1) * _N:, :]

    wc = wcol_ref[0]
    wr = wrow_ref[0]
    wxc = jax.lax.broadcast_in_dim(wc[:, 0:1], (_N, _N), (0, 1))
    wyc = jax.lax.broadcast_in_dim(wc[:, 1:2], (_N, _N), (0, 1))
    wxr = jax.lax.broadcast_in_dim(wr[0:1, :], (_N, _N), (0, 1))
    wyr = jax.lax.broadcast_in_dim(wr[1:2, :], (_N, _N), (0, 1))
    dx = wxc - wxr
    dy = wyc - wyr
    dist = jnp.sqrt(dx * dx + dy * dy + 1e-12)

    col = jax.lax.broadcasted_iota(jnp.int32, (_N, _N), 1)
    cur = dist
    mask = jnp.zeros((_N, _N), jnp.bool_)
    for _ in range(_K):
        rmin = jnp.min(cur, axis=1, keepdims=True)
        key = jnp.where(cur == rmin, col, _N)
        sidx = jnp.min(key, axis=1, keepdims=True)
        sel = col == sidx
        mask = jnp.logical_or(mask, sel)
        cur = jnp.where(sel, _BIG, cur)

    q = _dotT(he_last, wq_ref[...]) + bq_ref[...]
    qk = _dot(q, wk_ref[...])
    qb = jnp.sum(q * bk_ref[...], axis=1, keepdims=True)

    s = _dotT(qk, hist)
    s = (s + qb) * (1.0 / math.sqrt(_D))
    mask3 = jnp.concatenate([mask] * _TAU, axis=1)
    sm = jnp.where(mask3, s, -_BIG)
    mrow = jnp.max(sm, axis=1, keepdims=True)
    e = jnp.where(mask3, jnp.exp(s - mrow), 0.0)
    den = jnp.sum(e, axis=1, keepdims=True)
    w = e / den

    ctx = _dot(w, hist)
    ctx = _dotT(ctx, wv_ref[...]) + bv_ref[...]
    h1 = _dotT(ctx, w1_ref[...]) + b1_ref[...]
    g = 0.5 * h1 * (1.0 + _erf(h1 * (1.0 / math.sqrt(2.0))))
    out_ref[0] = _dotT(g, w2_ref[...]) + b2_ref[...]


@jax.jit
def kernel(h_e, x_orig, Wq, bq, Wk, bk, Wv, bv, W1, b1, W2, b2):
    b, seq_len, n, d = h_e.shape
    t0 = seq_len - 1
    t_start = max(0, t0 - _TAU + 1)
    tau_eff = t0 - t_start + 1
    hist = h_e[:, t_start:t0 + 1].reshape(b, tau_eff * n, d)

    last_wind = x_orig[t0, :, :, 4:6]
    wcol = jnp.pad(last_wind, ((0, 0), (0, 0), (0, 128 - 2)))
    wrow = jnp.pad(jnp.transpose(last_wind, (0, 2, 1)), ((0, 0), (0, 6), (0, 0)))

    full = lambda shape: pl.BlockSpec(shape, lambda i: (0,) * len(shape))
    per_b = lambda shape: pl.BlockSpec((1,) + shape, lambda i: (i, 0, 0))

    out = pl.pallas_call(
        _body,
        grid=(b,),
        in_specs=[
            per_b((tau_eff * n, d)),
            per_b((n, 128)),
            per_b((8, n)),
            full((d, d)), full((d, d)), full((d, d)), full((d, d)), full((d, d)),
            full((1, d)), full((1, d)), full((1, d)), full((1, d)), full((1, d)),
        ],
        out_specs=per_b((n, d)),
        out_shape=jax.ShapeDtypeStruct((b, n, d), jnp.float32),
    )(hist, wcol, wrow, Wq, Wk, Wv, W1, W2,
      bq.reshape(1, d), bk.reshape(1, d), bv.reshape(1, d),
      b1.reshape(1, d), b2.reshape(1, d))
    return out


# speedup vs baseline: 18.9800x; 1.4959x over previous
import functools
import math

import jax
import jax.numpy as jnp
from jax.experimental import pallas as pl

_BATCH = 16
_N = 512
_D = 128
_TAU = 3
_K = 8
_BIG = 3.0e38


def _dotT(a, b):
    return jax.lax.dot_general(a, b, (((1,), (1,)), ((), ())),
                               preferred_element_type=jnp.float32)


def _dot(a, b):
    return jax.lax.dot_general(a, b, (((1,), (0,)), ((), ())),
                               preferred_element_type=jnp.float32)


def _erf(x):
    a1, a2, a3 = 0.254829592, -0.284496736, 1.421413741
    a4, a5, p = -1.453152027, 1.061405429, 0.3275911
    s = jnp.sign(x)
    ax = jnp.abs(x)
    t = 1.0 / (1.0 + p * ax)
    y = 1.0 - (((((a5 * t + a4) * t) + a3) * t + a2) * t + a1) * t * jnp.exp(-ax * ax)
    return s * y


def _body(hist_ref, wcol_ref, wrow_ref,
          wq_ref, wk_ref, wv_ref, w1_ref, w2_ref,
          bq_ref, bk_ref, bv_ref, b1_ref, b2_ref,
          out_ref):
    hist = hist_ref[0]
    he_last = hist[(_TAU - 1) * _N:, :]

    wc = wcol_ref[0]
    wr = wrow_ref[0]
    wxc = jax.lax.broadcast_in_dim(wc[:, 0:1], (_N, _N), (0, 1))
    wyc = jax.lax.broadcast_in_dim(wc[:, 1:2], (_N, _N), (0, 1))
    wxr = jax.lax.broadcast_in_dim(wr[0:1, :], (_N, _N), (0, 1))
    wyr = jax.lax.broadcast_in_dim(wr[1:2, :], (_N, _N), (0, 1))
    dx = wxc - wxr
    dy = wyc - wyr
    d2 = dx * dx + dy * dy

    cur = d2
    for _ in range(_K):
        rmin = jnp.min(cur, axis=1, keepdims=True)
        cur = jnp.where(cur == rmin, _BIG, cur)
    mask = cur > d2

    q = _dotT(he_last, wq_ref[...]) + bq_ref[...]
    qk = _dot(q, wk_ref[...])

    s = _dotT(qk, hist)
    mask3 = jnp.concatenate([mask] * _TAU, axis=1)
    mrow = jnp.max(s, axis=1, keepdims=True)
    e = jnp.where(mask3, jnp.exp((s - mrow) * (1.0 / math.sqrt(_D))), 0.0)
    den = jnp.sum(e, axis=1, keepdims=True)

    ctx = _dot(e, hist) / den
    ctx = _dotT(ctx, wv_ref[...]) + bv_ref[...]
    h1 = _dotT(ctx, w1_ref[...]) + b1_ref[...]
    g = 0.5 * h1 * (1.0 + _erf(h1 * (1.0 / math.sqrt(2.0))))
    out_ref[0] = _dotT(g, w2_ref[...]) + b2_ref[...]


@jax.jit
def kernel(h_e, x_orig, Wq, bq, Wk, bk, Wv, bv, W1, b1, W2, b2):
    b, seq_len, n, d = h_e.shape
    t0 = seq_len - 1
    t_start = max(0, t0 - _TAU + 1)
    tau_eff = t0 - t_start + 1
    hist = h_e[:, t_start:t0 + 1].reshape(b, tau_eff * n, d)

    last_wind = x_orig[t0, :, :, 4:6]
    wcol = jnp.pad(last_wind, ((0, 0), (0, 0), (0, 128 - 2)))
    wrow = jnp.pad(jnp.transpose(last_wind, (0, 2, 1)), ((0, 0), (0, 6), (0, 0)))

    full = lambda shape: pl.BlockSpec(shape, lambda i: (0,) * len(shape))
    per_b = lambda shape: pl.BlockSpec((1,) + shape, lambda i: (i, 0, 0))

    out = pl.pallas_call(
        _body,
        grid=(b,),
        in_specs=[
            per_b((tau_eff * n, d)),
            per_b((n, 128)),
            per_b((8, n)),
            full((d, d)), full((d, d)), full((d, d)), full((d, d)), full((d, d)),
            full((1, d)), full((1, d)), full((1, d)), full((1, d)), full((1, d)),
        ],
        out_specs=per_b((n, d)),
        out_shape=jax.ShapeDtypeStruct((b, n, d), jnp.float32),
    )(hist, wcol, wrow, Wq, Wk, Wv, W1, W2,
      bq.reshape(1, d), bk.reshape(1, d), bv.reshape(1, d),
      b1.reshape(1, d), b2.reshape(1, d))
    return out
